# Initial kernel scaffold; baseline (speedup 1.0000x reference)
import jax
import jax.numpy as jnp
from jax.experimental import pallas as pl


def kernel(images, W, memory_bank, indices):
    raise NotImplementedError("write your pallas kernel here")


# speedup vs baseline: 1.5316x; 1.5316x over previous
import functools

import jax
import jax.numpy as jnp
from jax.experimental import pallas as pl
from jax.experimental.pallas import tpu as pltpu

_FEATURE = 128
_DATA = 262144
_TEMP = 0.07
_BATCH = 128

_KBLK = 3072
_NBLK = 4096


def _embed_body(nk, img_ref, w_ref, emb_ref, embT_ref, acc_ref):
    k = pl.program_id(0)

    @pl.when(k == 0)
    def _init():
        acc_ref[...] = jnp.zeros_like(acc_ref)

    acc_ref[...] += jnp.dot(img_ref[...], w_ref[...],
                            preferred_element_type=jnp.float32)

    @pl.when(k == nk - 1)
    def _finish():
        acc = acc_ref[...]
        norm = jnp.sqrt(jnp.sum(acc * acc, axis=1, keepdims=True)) + 1e-12
        emb = acc / norm
        emb_ref[...] = emb
        embT_ref[...] = emb.T


def _bank_body(nblk, emb_ref, embT_ref, idx_ref, bank_ref, logits_ref, nb_ref):
    bank = bank_ref[...]
    logits_ref[...] = jnp.dot(emb_ref[...], bank,
                              preferred_element_type=jnp.float32) * (1.0 / _TEMP)
    j = pl.program_id(0)
    cols = jax.lax.broadcasted_iota(jnp.int32, (_BATCH, nblk), 1) + j * nblk
    match = (idx_ref[...] == cols).astype(jnp.float32)
    sel = jax.lax.dot_general(embT_ref[...], match,
                              (((1,), (0,)), ((), ())),
                              preferred_element_type=jnp.float32)
    hit = jnp.max(match, axis=0, keepdims=True)
    nb_ref[...] = jnp.where(hit > 0.0, sel, bank)


def kernel(images, W, memory_bank, indices):
    feats = images.reshape(_BATCH, -1)
    kdim = feats.shape[1]
    nk = kdim // _KBLK

    emb, embT = pl.pallas_call(
        functools.partial(_embed_body, nk),
        grid=(nk,),
        in_specs=[
            pl.BlockSpec((_BATCH, _KBLK), lambda k: (0, k)),
            pl.BlockSpec((_KBLK, _FEATURE), lambda k: (k, 0)),
        ],
        out_specs=[
            pl.BlockSpec((_BATCH, _FEATURE), lambda k: (0, 0)),
            pl.BlockSpec((_FEATURE, _BATCH), lambda k: (0, 0)),
        ],
        out_shape=[
            jax.ShapeDtypeStruct((_BATCH, _FEATURE), jnp.float32),
            jax.ShapeDtypeStruct((_FEATURE, _BATCH), jnp.float32),
        ],
        scratch_shapes=[pltpu.VMEM((_BATCH, _FEATURE), jnp.float32)],
    )(feats, W)

    ar = jnp.arange(_BATCH)
    dup_later = jnp.any(
        (indices[None, :] == indices[:, None]) & (ar[None, :] > ar[:, None]),
        axis=1)
    scatter_idx = jnp.where(dup_later, -1, indices).reshape(_BATCH, 1)

    nj = _DATA // _NBLK
    logits, new_bank = pl.pallas_call(
        functools.partial(_bank_body, _NBLK),
        grid=(nj,),
        in_specs=[
            pl.BlockSpec((_BATCH, _FEATURE), lambda j: (0, 0)),
            pl.BlockSpec((_FEATURE, _BATCH), lambda j: (0, 0)),
            pl.BlockSpec((_BATCH, 1), lambda j: (0, 0)),
            pl.BlockSpec((_FEATURE, _NBLK), lambda j: (0, j)),
        ],
        out_specs=[
            pl.BlockSpec((_BATCH, _NBLK), lambda j: (0, j)),
            pl.BlockSpec((_FEATURE, _NBLK), lambda j: (0, j)),
        ],
        out_shape=[
            jax.ShapeDtypeStruct((_BATCH, _DATA), jnp.float32),
            jax.ShapeDtypeStruct((_FEATURE, _DATA), jnp.float32),
        ],
    )(emb, embT, scatter_idx, memory_bank)

    return (emb, logits, new_bank)
